# baseline (speedup 1.0000x reference)
import functools

import jax
import jax.numpy as jnp
from jax import lax
from jax.experimental import pallas as pl
from jax.experimental.pallas import tpu as pltpu, tpu_sc as plsc

R = 512
TI = 64
NUM_RBF = 32
MU_STEP = 20.0 / (NUM_RBF - 1)
INV_SIGMA = NUM_RBF / 20.0

_SC_INFO = plsc.get_sparse_core_info()
_NC = _SC_INFO.num_cores
_NS = _SC_INFO.num_subcores
_NW = _NC * _NS
_RPW = R // _NW
_L = 16
_NCHUNK = R // _L


def _fused_kernel(x_ref, c_ref, wn_ref, bn_ref, w_ref,
                  nh_ref, mcol_ref, mrow_ref, xc_ref, xct_ref, eh_ref):
    i = pl.program_id(0)

    @pl.when(i == 0)
    def _node_path():
        x = x_ref[...]
        dX = x[:, 1:, :] - x[:, :-1, :]
        l = jnp.sqrt(jnp.sum(dX * dX, axis=-1))
        log_len = jnp.log(l + 1e-6)
        u = dX / (l + 1e-6)[..., None]
        feat = jnp.concatenate(
            [log_len, u[:, 0, :], u[:, 1, :], u[:, 2, :]], axis=-1)
        m = (c_ref[...] > 0).astype(jnp.float32)
        nh = jnp.dot(feat, wn_ref[...], preferred_element_type=jnp.float32)
        nh_ref[...] = (nh + bn_ref[...]) * m
        mcol_ref[...] = m
        mrow_ref[...] = m.reshape(1, R)
        xc = jnp.mean(x, axis=1)
        xc_ref[...] = xc
        xct_ref[...] = xc.T

    base = i * TI
    xi = xc_ref[pl.ds(base, TI), :]
    xjt = xct_ref[...]
    dx = xjt[0:1, :] - xi[:, 0:1]
    dy = xjt[1:2, :] - xi[:, 1:2]
    dz = xjt[2:3, :] - xi[:, 2:3]
    m = mcol_ref[pl.ds(base, TI), :] * mrow_ref[...]
    d2 = dx * dx + dy * dy + dz * dz
    d = jnp.sqrt(d2)
    rinv = 1.0 / (d + 1e-6)
    uxm = dx * rinv * m
    uym = dy * rinv * m
    uzm = dz * rinv * m
    neg_big = (m - 1.0) * 1e30
    mu = jax.lax.broadcasted_iota(
        jnp.int32, (1, NUM_RBF, 1), 1).astype(jnp.float32) * MU_STEP
    t = (d[:, None, :] - mu) * INV_SIGMA
    rbf = jnp.exp(neg_big[:, None, :] - t * t)
    feat = jnp.concatenate(
        [rbf, uxm[:, None, :], uym[:, None, :], uzm[:, None, :],
         m[:, None, :]], axis=1)
    out = jax.lax.dot_general(
        feat, w_ref[...], (((1,), (0,)), ((), ())),
        preferred_element_type=jnp.float32)
    eh_ref[...] = out


def _sc_kernel(c_hbm, idx_hbm, mij_hbm, tbl_hbm,
               c_v, mf_v, zero_v, rowi_v, idxv_v, blkm_v, blki_v, sem, sem2):
    wid = lax.axis_index("s") * _NC + lax.axis_index("c")
    base = wid * _RPW
    pltpu.sync_copy(c_hbm, c_v)
    for j in range(_NCHUNK):
        sl = pl.ds(j * _L, _L)
        cj = c_v[sl]
        mf_v[sl] = jnp.where(cj > 0, 1.0, 0.0)
        zero_v[sl] = jnp.zeros((_L,), jnp.float32)
        rowi_v[sl] = lax.iota(jnp.int32, _L) + j * _L
    t1 = pltpu.async_copy(zero_v, tbl_hbm.at[2 * wid], sem)
    t2 = pltpu.async_copy(mf_v, tbl_hbm.at[2 * wid + 1], sem)
    myc = c_v[pl.ds(base, _RPW)]
    idxv_v[...] = jnp.where(myc > 0, 1, 0) + 2 * wid
    for i in range(_RPW):
        for j in range(_NCHUNK):
            sl = pl.ds(j * _L, _L)
            blki_v[i, sl] = rowi_v[sl]
    t3 = pltpu.async_copy(blki_v, idx_hbm.at[pl.ds(base, _RPW)], sem2)
    t1.wait()
    t2.wait()
    pltpu.async_copy(tbl_hbm.at[idxv_v], blkm_v, sem).wait()
    pltpu.sync_copy(blkm_v, mij_hbm.at[pl.ds(base, _RPW)])
    t3.wait()


_sc_call = functools.partial(
    pl.kernel,
    mesh=plsc.VectorSubcoreMesh(core_axis_name="c", subcore_axis_name="s"),
    out_type=(
        jax.ShapeDtypeStruct((R, R), jnp.int32),
        jax.ShapeDtypeStruct((R, R), jnp.float32),
        jax.ShapeDtypeStruct((2 * _NW, R), jnp.float32),
    ),
    scratch_types=[
        pltpu.VMEM((R,), jnp.int32),
        pltpu.VMEM((R,), jnp.float32),
        pltpu.VMEM((R,), jnp.float32),
        pltpu.VMEM((R,), jnp.int32),
        pltpu.VMEM((_RPW,), jnp.int32),
        pltpu.VMEM((_RPW, R), jnp.float32),
        pltpu.VMEM((_RPW, R), jnp.int32),
        pltpu.SemaphoreType.DMA,
        pltpu.SemaphoreType.DMA,
    ],
)(_sc_kernel)


def kernel(X, C, W_node, b_node, W_edge, b_edge):
    B = X.shape[0]
    x = X.reshape(R, 4, 3)
    c_col = C.reshape(R, 1)
    bn = b_node.reshape(1, -1)
    dim_nodes = W_node.shape[1]
    dim_edges = W_edge.shape[1]

    w36 = jnp.concatenate([W_edge, b_edge[None, :]], axis=0)

    nblk = R // TI
    const = lambda i: (0, 0)
    outs = pl.pallas_call(
        _fused_kernel,
        grid=(nblk,),
        in_specs=[
            pl.BlockSpec((R, 4, 3), lambda i: (0, 0, 0)),
            pl.BlockSpec((R, 1), const),
            pl.BlockSpec((12, dim_nodes), const),
            pl.BlockSpec((1, dim_nodes), const),
            pl.BlockSpec((NUM_RBF + 4, dim_edges), const),
        ],
        out_specs=(
            pl.BlockSpec((R, dim_nodes), const),
            pl.BlockSpec((R, 1), const),
            pl.BlockSpec((1, R), const),
            pl.BlockSpec((R, 3), const),
            pl.BlockSpec((3, R), const),
            pl.BlockSpec((TI, R, dim_edges), lambda i: (i, 0, 0)),
        ),
        out_shape=(
            jax.ShapeDtypeStruct((R, dim_nodes), jnp.float32),
            jax.ShapeDtypeStruct((R, 1), jnp.float32),
            jax.ShapeDtypeStruct((1, R), jnp.float32),
            jax.ShapeDtypeStruct((R, 3), jnp.float32),
            jax.ShapeDtypeStruct((3, R), jnp.float32),
            jax.ShapeDtypeStruct((R, R, dim_edges), jnp.float32),
        ),
    )(x, c_col, W_node, bn, w36)
    node_h, _mcol, mrow, _xc, _xct, edge_h = outs

    edge_idx, mask_ij, _tbl = _sc_call(C.reshape(R))

    return (node_h.reshape(B, R, dim_nodes),
            edge_h.reshape(B, R, R, dim_edges),
            edge_idx.reshape(B, R, R),
            mrow.reshape(B, R),
            mask_ij.reshape(B, R, R))


# speedup vs baseline: 1.1494x; 1.1494x over previous
import jax
import jax.numpy as jnp
from jax.experimental import pallas as pl

R = 512
TI = 16
NUM_RBF = 32
MU_STEP = 20.0 / (NUM_RBF - 1)
INV_SIGMA = NUM_RBF / 20.0


def _fused_kernel(x_ref, c_ref, wn_ref, bn_ref, w_ref,
                  nh_ref, mcol_ref, mrow_ref, xc_ref, xct_ref,
                  eh_ref, mij_ref, idx_ref):
    i = pl.program_id(0)

    @pl.when(i == 0)
    def _node_path():
        x = x_ref[...]
        dX = x[:, 1:, :] - x[:, :-1, :]
        l = jnp.sqrt(jnp.sum(dX * dX, axis=-1))
        log_len = jnp.log(l + 1e-6)
        u = dX / (l + 1e-6)[..., None]
        feat = jnp.concatenate(
            [log_len, u[:, 0, :], u[:, 1, :], u[:, 2, :]], axis=-1)
        m = (c_ref[...] > 0).astype(jnp.float32)
        nh = jnp.dot(feat, wn_ref[...], preferred_element_type=jnp.float32)
        nh_ref[...] = (nh + bn_ref[...]) * m
        mcol_ref[...] = m
        mrow_ref[...] = m.reshape(1, R)
        xc = jnp.mean(x, axis=1)
        xc_ref[...] = xc
        xct_ref[...] = xc.T

    base = i * TI
    xi = xc_ref[pl.ds(base, TI), :]
    xjt = xct_ref[...]
    dx = xjt[0:1, :] - xi[:, 0:1]
    dy = xjt[1:2, :] - xi[:, 1:2]
    dz = xjt[2:3, :] - xi[:, 2:3]
    m = mcol_ref[pl.ds(base, TI), :] * mrow_ref[...]
    mij_ref[...] = m
    idx_ref[...] = jax.lax.broadcasted_iota(jnp.int32, (TI, R), 1)
    d2 = dx * dx + dy * dy + dz * dz
    d = jnp.sqrt(d2)
    rinv = 1.0 / (d + 1e-6)
    uxm = dx * rinv * m
    uym = dy * rinv * m
    uzm = dz * rinv * m
    neg_big = (m - 1.0) * 1e30
    mu = jax.lax.broadcasted_iota(
        jnp.int32, (1, NUM_RBF, 1), 1).astype(jnp.float32) * MU_STEP
    t = (d[:, None, :] - mu) * INV_SIGMA
    rbf = jnp.exp(neg_big[:, None, :] - t * t)
    feat = jnp.concatenate(
        [rbf, uxm[:, None, :], uym[:, None, :], uzm[:, None, :],
         m[:, None, :]], axis=1)
    out = jax.lax.dot_general(
        feat, w_ref[...], (((1,), (0,)), ((), ())),
        preferred_element_type=jnp.float32)
    eh_ref[...] = out


def kernel(X, C, W_node, b_node, W_edge, b_edge):
    B = X.shape[0]
    x = X.reshape(R, 4, 3)
    c_col = C.reshape(R, 1)
    bn = b_node.reshape(1, -1)
    dim_nodes = W_node.shape[1]
    dim_edges = W_edge.shape[1]

    w36 = jnp.concatenate([W_edge, b_edge[None, :]], axis=0)

    nblk = R // TI
    const = lambda i: (0, 0)
    outs = pl.pallas_call(
        _fused_kernel,
        grid=(nblk,),
        in_specs=[
            pl.BlockSpec((R, 4, 3), lambda i: (0, 0, 0)),
            pl.BlockSpec((R, 1), const),
            pl.BlockSpec((12, dim_nodes), const),
            pl.BlockSpec((1, dim_nodes), const),
            pl.BlockSpec((NUM_RBF + 4, dim_edges), const),
        ],
        out_specs=(
            pl.BlockSpec((R, dim_nodes), const),
            pl.BlockSpec((R, 1), const),
            pl.BlockSpec((1, R), const),
            pl.BlockSpec((R, 3), const),
            pl.BlockSpec((3, R), const),
            pl.BlockSpec((TI, R, dim_edges), lambda i: (i, 0, 0)),
            pl.BlockSpec((TI, R), lambda i: (i, 0)),
            pl.BlockSpec((TI, R), lambda i: (i, 0)),
        ),
        out_shape=(
            jax.ShapeDtypeStruct((R, dim_nodes), jnp.float32),
            jax.ShapeDtypeStruct((R, 1), jnp.float32),
            jax.ShapeDtypeStruct((1, R), jnp.float32),
            jax.ShapeDtypeStruct((R, 3), jnp.float32),
            jax.ShapeDtypeStruct((3, R), jnp.float32),
            jax.ShapeDtypeStruct((R, R, dim_edges), jnp.float32),
            jax.ShapeDtypeStruct((R, R), jnp.float32),
            jax.ShapeDtypeStruct((R, R), jnp.int32),
        ),
    )(x, c_col, W_node, bn, w36)
    node_h, _mcol, mrow, _xc, _xct, edge_h, mask_ij, edge_idx = outs

    return (node_h.reshape(B, R, dim_nodes),
            edge_h.reshape(B, R, R, dim_edges),
            edge_idx.reshape(B, R, R),
            mrow.reshape(B, R),
            mask_ij.reshape(B, R, R))


# speedup vs baseline: 1.3114x; 1.1410x over previous
import jax
import jax.numpy as jnp
from jax.experimental import pallas as pl

R = 512
TI = 64
NUM_RBF = 32
MU_STEP = 20.0 / (NUM_RBF - 1)
INV_SIGMA = NUM_RBF / 20.0


def _fused_kernel(x_ref, c_ref, wn_ref, bn_ref, we_ref, be_ref,
                  nh_ref, mcol_ref, mrow_ref, xc_ref, xct_ref,
                  eh_ref, mij_ref, idx_ref):
    i = pl.program_id(0)

    @pl.when(i == 0)
    def _node_path():
        x = x_ref[...]
        dX = x[:, 1:, :] - x[:, :-1, :]
        l = jnp.sqrt(jnp.sum(dX * dX, axis=-1))
        log_len = jnp.log(l + 1e-6)
        u = dX / (l + 1e-6)[..., None]
        feat = jnp.concatenate(
            [log_len, u[:, 0, :], u[:, 1, :], u[:, 2, :]], axis=-1)
        m = (c_ref[...] > 0).astype(jnp.float32)
        nh = jnp.dot(feat, wn_ref[...], preferred_element_type=jnp.float32)
        nh_ref[...] = (nh + bn_ref[...]) * m
        mcol_ref[...] = m
        mrow_ref[...] = m.reshape(1, R)
        xc = jnp.mean(x, axis=1)
        xc_ref[...] = xc
        xct_ref[...] = xc.T

    base = i * TI
    xi = xc_ref[pl.ds(base, TI), :]
    xjt = xct_ref[...]
    dx = xjt[0:1, :] - xi[:, 0:1]
    dy = xjt[1:2, :] - xi[:, 1:2]
    dz = xjt[2:3, :] - xi[:, 2:3]
    m = mcol_ref[pl.ds(base, TI), :] * mrow_ref[...]
    mij_ref[...] = m
    idx_ref[...] = jax.lax.broadcasted_iota(jnp.int32, (TI, R), 1)
    d2 = dx * dx + dy * dy + dz * dz
    d = jnp.sqrt(d2)
    rinv = 1.0 / (d + 1e-6)
    uxm = dx * rinv * m
    uym = dy * rinv * m
    uzm = dz * rinv * m
    neg_big = (m - 1.0) * 1e30
    mu = jax.lax.broadcasted_iota(
        jnp.int32, (1, NUM_RBF, 1), 1).astype(jnp.float32) * MU_STEP
    t = (d[:, None, :] - mu) * INV_SIGMA
    rbf = jnp.exp(neg_big[:, None, :] - t * t)
    feat = jnp.concatenate(
        [rbf, uxm[:, None, :], uym[:, None, :], uzm[:, None, :],
         m[:, None, :]], axis=1)
    w36 = jnp.concatenate([we_ref[...], be_ref[...]], axis=0)
    out = jax.lax.dot_general(
        feat, w36, (((1,), (0,)), ((), ())),
        preferred_element_type=jnp.float32)
    eh_ref[...] = out


def kernel(X, C, W_node, b_node, W_edge, b_edge):
    B = X.shape[0]
    x = X.reshape(R, 4, 3)
    c_col = C.reshape(R, 1)
    bn = b_node.reshape(1, -1)
    dim_nodes = W_node.shape[1]
    dim_edges = W_edge.shape[1]

    be = b_edge.reshape(1, -1)

    nblk = R // TI
    const = lambda i: (0, 0)
    outs = pl.pallas_call(
        _fused_kernel,
        grid=(nblk,),
        in_specs=[
            pl.BlockSpec((R, 4, 3), lambda i: (0, 0, 0)),
            pl.BlockSpec((R, 1), const),
            pl.BlockSpec((12, dim_nodes), const),
            pl.BlockSpec((1, dim_nodes), const),
            pl.BlockSpec((NUM_RBF + 3, dim_edges), const),
            pl.BlockSpec((1, dim_edges), const),
        ],
        out_specs=(
            pl.BlockSpec((R, dim_nodes), const),
            pl.BlockSpec((R, 1), const),
            pl.BlockSpec((1, R), const),
            pl.BlockSpec((R, 3), const),
            pl.BlockSpec((3, R), const),
            pl.BlockSpec((TI, R, dim_edges), lambda i: (i, 0, 0)),
            pl.BlockSpec((TI, R), lambda i: (i, 0)),
            pl.BlockSpec((TI, R), lambda i: (i, 0)),
        ),
        out_shape=(
            jax.ShapeDtypeStruct((R, dim_nodes), jnp.float32),
            jax.ShapeDtypeStruct((R, 1), jnp.float32),
            jax.ShapeDtypeStruct((1, R), jnp.float32),
            jax.ShapeDtypeStruct((R, 3), jnp.float32),
            jax.ShapeDtypeStruct((3, R), jnp.float32),
            jax.ShapeDtypeStruct((R, R, dim_edges), jnp.float32),
            jax.ShapeDtypeStruct((R, R), jnp.float32),
            jax.ShapeDtypeStruct((R, R), jnp.int32),
        ),
    )(x, c_col, W_node, bn, W_edge, be)
    node_h, _mcol, mrow, _xc, _xct, edge_h, mask_ij, edge_idx = outs

    return (node_h.reshape(B, R, dim_nodes),
            edge_h.reshape(B, R, R, dim_edges),
            edge_idx.reshape(B, R, R),
            mrow.reshape(B, R),
            mask_ij.reshape(B, R, R))


# speedup vs baseline: 1.3960x; 1.0645x over previous
import jax
import jax.numpy as jnp
from jax.experimental import pallas as pl
from jax.experimental.pallas import tpu as pltpu

R = 512
TI = 64
NUM_RBF = 32
MU_STEP = 20.0 / (NUM_RBF - 1)
INV_SIGMA = NUM_RBF / 20.0


def _fused_kernel(x12_ref, ccol_ref, crow_ref, wn_ref, bn_ref, we_ref, be_ref,
                  nh_ref, mcol_ref, mrow_ref, xc_ref, xct_ref,
                  eh_ref, mij_ref, idx_ref, xt_ref):
    i = pl.program_id(0)
    nsteps = pl.num_programs(0)

    @pl.when(i == 0)
    def _prep():
        xt = jnp.transpose(x12_ref[...])
        xt_ref[...] = xt
        xct = (xt[0:3, :] + xt[3:6, :] + xt[6:9, :] + xt[9:12, :]) * 0.25
        xct_ref[...] = xct
        xc_ref[...] = xct.T
        mcol_ref[...] = (ccol_ref[...] > 0).astype(jnp.float32)
        mrow_ref[...] = (crow_ref[...] > 0).astype(jnp.float32)

    @pl.when(i == nsteps - 1)
    def _node_path():
        xt = xt_ref[...]
        dxt = xt[3:12, :] - xt[0:9, :]
        sq = dxt * dxt
        l = jnp.sqrt(jnp.concatenate(
            [sq[0:1] + sq[1:2] + sq[2:3],
             sq[3:4] + sq[4:5] + sq[5:6],
             sq[6:7] + sq[7:8] + sq[8:9]], axis=0))
        log_len = jnp.log(l + 1e-6)
        inv = 1.0 / (l + 1e-6)
        inv9 = jnp.concatenate(
            [inv[0:1], inv[0:1], inv[0:1],
             inv[1:2], inv[1:2], inv[1:2],
             inv[2:3], inv[2:3], inv[2:3]], axis=0)
        featt = jnp.concatenate([log_len, dxt * inv9], axis=0)
        nh = jax.lax.dot_general(
            featt, wn_ref[...], (((0,), (0,)), ((), ())),
            preferred_element_type=jnp.float32)
        nh_ref[...] = (nh + bn_ref[...]) * mcol_ref[...]

    base = i * TI
    xi = xc_ref[pl.ds(base, TI), :]
    xjt = xct_ref[...]
    dx = xjt[0:1, :] - xi[:, 0:1]
    dy = xjt[1:2, :] - xi[:, 1:2]
    dz = xjt[2:3, :] - xi[:, 2:3]
    m = mcol_ref[pl.ds(base, TI), :] * mrow_ref[...]
    mij_ref[...] = m
    idx_ref[...] = jax.lax.broadcasted_iota(jnp.int32, (TI, R), 1)
    d2 = dx * dx + dy * dy + dz * dz
    d = jnp.sqrt(d2)
    rinv = 1.0 / (d + 1e-6)
    uxm = dx * rinv * m
    uym = dy * rinv * m
    uzm = dz * rinv * m
    neg_big = (m - 1.0) * 1e30
    mu = jax.lax.broadcasted_iota(
        jnp.int32, (1, NUM_RBF, 1), 1).astype(jnp.float32) * MU_STEP
    t = (d[:, None, :] - mu) * INV_SIGMA
    rbf = jnp.exp(neg_big[:, None, :] - t * t)
    feat = jnp.concatenate(
        [rbf, uxm[:, None, :], uym[:, None, :], uzm[:, None, :],
         m[:, None, :]], axis=1)
    w36 = jnp.concatenate([we_ref[...], be_ref[...]], axis=0)
    out = jax.lax.dot_general(
        feat, w36, (((1,), (0,)), ((), ())),
        preferred_element_type=jnp.float32)
    eh_ref[...] = out


def kernel(X, C, W_node, b_node, W_edge, b_edge):
    B = X.shape[0]
    x12 = X.reshape(R, 12)
    c_col = C.reshape(R, 1)
    c_row = C.reshape(1, R)
    bn = b_node.reshape(1, -1)
    be = b_edge.reshape(1, -1)
    dim_nodes = W_node.shape[1]
    dim_edges = W_edge.shape[1]

    nblk = R // TI
    const = lambda i: (0, 0)
    outs = pl.pallas_call(
        _fused_kernel,
        grid=(nblk,),
        in_specs=[
            pl.BlockSpec((R, 12), const),
            pl.BlockSpec((R, 1), const),
            pl.BlockSpec((1, R), const),
            pl.BlockSpec((12, dim_nodes), const),
            pl.BlockSpec((1, dim_nodes), const),
            pl.BlockSpec((NUM_RBF + 3, dim_edges), const),
            pl.BlockSpec((1, dim_edges), const),
        ],
        out_specs=(
            pl.BlockSpec((R, dim_nodes), const),
            pl.BlockSpec((R, 1), const),
            pl.BlockSpec((1, R), const),
            pl.BlockSpec((R, 3), const),
            pl.BlockSpec((3, R), const),
            pl.BlockSpec((TI, R, dim_edges), lambda i: (i, 0, 0)),
            pl.BlockSpec((TI, R), lambda i: (i, 0)),
            pl.BlockSpec((TI, R), lambda i: (i, 0)),
        ),
        out_shape=(
            jax.ShapeDtypeStruct((R, dim_nodes), jnp.float32),
            jax.ShapeDtypeStruct((R, 1), jnp.float32),
            jax.ShapeDtypeStruct((1, R), jnp.float32),
            jax.ShapeDtypeStruct((R, 3), jnp.float32),
            jax.ShapeDtypeStruct((3, R), jnp.float32),
            jax.ShapeDtypeStruct((R, R, dim_edges), jnp.float32),
            jax.ShapeDtypeStruct((R, R), jnp.float32),
            jax.ShapeDtypeStruct((R, R), jnp.int32),
        ),
        scratch_shapes=[pltpu.VMEM((12, R), jnp.float32)],
    )(x12, c_col, c_row, W_node, bn, W_edge, be)
    node_h, _mcol, mrow, _xc, _xct, edge_h, mask_ij, edge_idx = outs

    return (node_h.reshape(B, R, dim_nodes),
            edge_h.reshape(B, R, R, dim_edges),
            edge_idx.reshape(B, R, R),
            mrow.reshape(B, R),
            mask_ij.reshape(B, R, R))
